# baseline (speedup 1.0000x reference)
import functools

import jax
import jax.numpy as jnp
from jax import lax
from jax.experimental import pallas as pl
from jax.experimental.pallas import tpu as pltpu
from jax.experimental.pallas import tpu_sc as plsc

U = 100000
I = 100000
D = 128
B = 4096
T = 819200

NC = 2
NS = 16
NW = NC * NS
K = 64

ITEMS_PER_W = T // NW
CHUNKS_PER_W = ITEMS_PER_W // K
NBUF = 8
SLACK = 3
NRUN = 8
ROWS_PER_TILE = B // NS
SAMPLES_PER_W = B // NW

def _build_segsum_kernel():
    mesh = plsc.VectorSubcoreMesh(core_axis_name="c", subcore_axis_name="s")
    return functools.partial(
        pl.kernel,
        out_type=[
            jax.ShapeDtypeStruct((NC, B, D), jnp.float32),
            jax.ShapeDtypeStruct((NC, B), jnp.float32),
        ],
        mesh=mesh,
        compiler_params=pltpu.CompilerParams(needs_layout_passes=False),
        scratch_types=[
            pltpu.VMEM((3, NBUF, K), jnp.int32),
            pltpu.VMEM((3, NBUF, K), jnp.int32),
            [pltpu.VMEM((K, D), jnp.float32) for _ in range(NBUF)],
            pltpu.VMEM((2, NRUN, D), jnp.float32),
            pltpu.VMEM((2, NRUN), jnp.int32),
            pltpu.VMEM((K,), jnp.int32),
            pltpu.VMEM((ROWS_PER_TILE,), jnp.float32),
            pltpu.VMEM((K,), jnp.float32),
            pltpu.VMEM_SHARED((B, D), jnp.float32),
            pltpu.VMEM_SHARED((B,), jnp.float32),
            [pltpu.SemaphoreType.DMA for _ in range(NBUF)],
            [pltpu.SemaphoreType.DMA for _ in range(NBUF)],
            [pltpu.SemaphoreType.DMA for _ in range(2)],
            pltpu.SemaphoreType.DMA,
            pltpu.SemaphoreType.DMA,
        ],
    )(_segsum_body)


def _segsum_body(hist_items, hist_segments, impl_emb, partial_out,
                 counts_out, items_v, segs_v, rows, runsum_v, runseg_v,
                 runrow_v, zc_v, ones_v, accum_s, counts_s, gsem, csem,
                 rsem, isem, ssem):
    cid = lax.axis_index("c")
    sid = lax.axis_index("s")
    wid = cid * NS + sid
    ngroups = CHUNKS_PER_W // NBUF

    zeros16 = jnp.zeros((16,), jnp.float32)
    izeros16 = jnp.zeros((16,), jnp.int32)
    lanes = lax.iota(jnp.int32, 16)

    def _zero_row(r, _):
        for j in range(D // 16):
            rows[0][r, pl.ds(j * 16, 16)] = zeros16
        return 0

    lax.fori_loop(0, K, _zero_row, 0)
    for j in range(ROWS_PER_TILE // 16):
        zc_v[pl.ds(j * 16, 16)] = zeros16
    for j in range(K // 16):
        ones_v[pl.ds(j * 16, 16)] = jnp.ones((16,), jnp.float32)
    for pp in range(2):
        runseg_v[pp, pl.ds(0, 16)] = izeros16
        for r in range(NRUN):
            for j in range(D // 16):
                runsum_v[pp, r, pl.ds(j * 16, 16)] = zeros16

    row0 = pl.multiple_of(sid * ROWS_PER_TILE, ROWS_PER_TILE)
    for h in range(ROWS_PER_TILE // K):
        pltpu.sync_copy(rows[0], accum_s.at[pl.ds(row0 + h * K, K)])
    pltpu.sync_copy(zc_v, counts_s.at[pl.ds(row0, ROWS_PER_TILE)])
    plsc.subcore_barrier()

    chunk0 = pl.multiple_of(wid * CHUNKS_PER_W, CHUNKS_PER_W)

    def _idx_copies(g, slot):
        src_rows = pl.ds(chunk0 + g * NBUF, NBUF)
        return (pltpu.make_async_copy(hist_items.at[src_rows],
                                      items_v.at[slot], isem),
                pltpu.make_async_copy(hist_segments.at[src_rows],
                                      segs_v.at[slot], ssem))

    def _gather(slot, b, buf):
        return pltpu.make_async_copy(impl_emb.at[items_v.at[slot, b]],
                                     rows[buf], gsem[buf])

    def _count(slot, b):
        return pltpu.make_async_copy(ones_v,
                                     counts_s.at[segs_v.at[slot, b]],
                                     csem[b])

    def _run_scatter(pp):
        return pltpu.make_async_copy(runsum_v.at[pp],
                                     accum_s.at[runseg_v.at[pp]],
                                     rsem[pp])

    def _reduce_and_scatter(g, p, b):
        pp = b & 1
        rot_idx = jnp.bitwise_and(lanes + 15, 15)
        last_idx = izeros16 + 15

        carry_seg = izeros16 - 1
        base = izeros16
        bnds = []
        ridxs = []
        segs16 = []
        for i in range(K // 16):
            v = segs_v[p, b, pl.ds(i * 16, 16)]
            rot = v.at[rot_idx].get(mode="promise_in_bounds")
            prev = jnp.where(lanes == 0, carry_seg, rot)
            bnd = v != prev
            ridx = plsc.cumsum(bnd.astype(jnp.int32)) + base - 1
            ridx_c = jnp.minimum(ridx, NRUN - 1)
            runrow_v[pl.ds(i * 16, 16)] = ridx_c
            plsc.store_scatter(runseg_v,
                               [izeros16 + pp, ridx_c], v, mask=bnd)
            carry_seg = v.at[last_idx].get(mode="promise_in_bounds")
            base = ridx.at[last_idx].get(mode="promise_in_bounds") + 1
        nruns = jnp.max(base)
        overflow = nruns > NRUN

        for r in range(NRUN):
            for j in range(D // 16):
                runsum_v[pp, r, pl.ds(j * 16, 16)] = zeros16

        pp_vec = izeros16 + pp

        def _row(r, _):
            r0 = pl.multiple_of(jnp.bitwise_and(r, -16), 16)
            rvec = runrow_v[pl.ds(r0, 16)]
            ridx_splat = rvec.at[izeros16 + jnp.bitwise_and(r, 15)].get(
                mode="promise_in_bounds")
            for j in range(D // 16):
                x = rows[b][r, pl.ds(j * 16, 16)]
                plsc.addupdate_scatter(
                    runsum_v, [pp_vec, ridx_splat, lanes + j * 16], x)
            return 0

        lax.fori_loop(0, K, _row, 0)

        @pl.when(overflow)
        def _():
            for r in range(NRUN):
                for j in range(D // 16):
                    runsum_v[pp, r, pl.ds(j * 16, 16)] = zeros16
            pltpu.sync_copy(rows[b], accum_s.at[segs_v.at[p, b]],
                            add=True)

        _run_scatter(pp).start(add=True)

    for c in _idx_copies(0, 0):
        c.start()
    for c in _idx_copies(0, 0):
        c.wait()
    for b in range(NBUF - SLACK):
        _gather(0, b, b).start()
    for c in _idx_copies(1, 1):
        c.start()

    def _group(g, _):
        p = lax.rem(g, 3)
        pn = lax.rem(g + 1, 3)
        pm = lax.rem(g + 2, 3)

        for b in range(NBUF):
            _gather(p, b, b).wait()
            _count(p, b).start(add=True)

            if b >= 2:
                _run_scatter(b & 1).wait()
            else:
                @pl.when(g > 0)
                def _():
                    _run_scatter(b & 1).wait()

            _reduce_and_scatter(g, p, b)

            if b >= SLACK:
                bb = b - SLACK

                @pl.when(g < ngroups - 1)
                def _():
                    _gather(pn, bb, bb).start()
            else:
                bb = b + NBUF - SLACK
                _gather(p, bb, bb).start()

            if b == 1:
                @pl.when(g < ngroups - 1)
                def _():
                    for c in _idx_copies(g + 1, pn):
                        c.wait()

        for b in range(NBUF):
            _count(p, b).wait()

        @pl.when(g < ngroups - 2)
        def _():
            for c in _idx_copies(g + 2, pm):
                c.start()

        return 0

    lax.fori_loop(0, ngroups, _group, 0)
    for pp in range(2):
        _run_scatter(pp).wait()
    plsc.subcore_barrier()

    for h in range(ROWS_PER_TILE // K):
        pltpu.sync_copy(accum_s.at[pl.ds(row0 + h * K, K)],
                        partial_out.at[cid, pl.ds(row0 + h * K, K)])
    pltpu.sync_copy(counts_s.at[pl.ds(row0, ROWS_PER_TILE)],
                    counts_out.at[cid, pl.ds(row0, ROWS_PER_TILE)])


def _build_combine_kernel():
    mesh = plsc.VectorSubcoreMesh(core_axis_name="c", subcore_axis_name="s")
    return functools.partial(
        pl.kernel,
        out_type=jax.ShapeDtypeStruct((B,), jnp.float32),
        mesh=mesh,
        compiler_params=pltpu.CompilerParams(needs_layout_passes=False),
        scratch_types=[
            pltpu.VMEM((SAMPLES_PER_W,), jnp.int32),
            pltpu.VMEM((SAMPLES_PER_W,), jnp.int32),
            pltpu.VMEM((SAMPLES_PER_W, D), jnp.float32),
            pltpu.VMEM((SAMPLES_PER_W, D), jnp.float32),
            pltpu.VMEM((SAMPLES_PER_W, D), jnp.float32),
            pltpu.VMEM((SAMPLES_PER_W, D), jnp.float32),
            pltpu.VMEM((SAMPLES_PER_W,), jnp.float32),
            pltpu.VMEM((SAMPLES_PER_W,), jnp.float32),
            pltpu.VMEM((SAMPLES_PER_W,), jnp.float32),
            pltpu.VMEM((SAMPLES_PER_W,), jnp.float32),
            pltpu.VMEM((SAMPLES_PER_W,), jnp.float32),
            pltpu.SemaphoreType.DMA,
        ],
    )(_combine_body)


def _combine_body(user_ids, item_ids, user_emb, item_emb, user_bias,
                    item_bias, partial, counts, out,
                    uidx_v, iidx_v, ue_v, ie_v, p0_v, p1_v, ub_v, ib_v,
                    c0_v, c1_v, out_v, sem):
    cid = lax.axis_index("c")
    sid = lax.axis_index("s")
    wid = cid * NS + sid
    base = pl.multiple_of(wid * SAMPLES_PER_W, SAMPLES_PER_W)

    pltpu.sync_copy(user_ids.at[pl.ds(base, SAMPLES_PER_W)], uidx_v)
    pltpu.sync_copy(item_ids.at[pl.ds(base, SAMPLES_PER_W)], iidx_v)
    pltpu.async_copy(user_emb.at[uidx_v], ue_v, sem).wait()
    pltpu.async_copy(item_emb.at[iidx_v], ie_v, sem).wait()
    pltpu.async_copy(user_bias.at[uidx_v], ub_v, sem).wait()
    pltpu.async_copy(item_bias.at[iidx_v], ib_v, sem).wait()
    pltpu.sync_copy(partial.at[0, pl.ds(base, SAMPLES_PER_W)], p0_v)
    pltpu.sync_copy(partial.at[1, pl.ds(base, SAMPLES_PER_W)], p1_v)
    pltpu.sync_copy(counts.at[0, pl.ds(base, SAMPLES_PER_W)], c0_v)
    pltpu.sync_copy(counts.at[1, pl.ds(base, SAMPLES_PER_W)], c1_v)

    lanes = lax.iota(jnp.int32, 16)

    for g in range(SAMPLES_PER_W // 16):
        s0 = g * 16
        c = c0_v[pl.ds(s0, 16)] + c1_v[pl.ds(s0, 16)]
        x = jnp.maximum(c, 1.0)
        s = 0.5 * (x + 1.0)
        for _ in range(22):
            s = 0.5 * (s + x / s)
        norm = jnp.where(c > 0.0, 1.0 / s, 0.0)

        def _row(r, carry):
            a_vec, s_vec = carry
            a_acc = jnp.zeros((16,), jnp.float32)
            s_acc = jnp.zeros((16,), jnp.float32)
            row = s0 + r
            for j in range(D // 16):
                sl = pl.ds(j * 16, 16)
                iv = ie_v[row, sl]
                a_acc = a_acc + ue_v[row, sl] * iv
                s_acc = s_acc + (p0_v[row, sl] + p1_v[row, sl]) * iv
            sel = lanes == r
            a_vec = jnp.where(sel, jnp.sum(a_acc), a_vec)
            s_vec = jnp.where(sel, jnp.sum(s_acc), s_vec)
            return a_vec, s_vec

        zero16 = jnp.zeros((16,), jnp.float32)
        a_vec, s_vec = lax.fori_loop(0, 16, _row, (zero16, zero16))
        out_v[pl.ds(s0, 16)] = (ub_v[pl.ds(s0, 16)] + ib_v[pl.ds(s0, 16)]
                                + a_vec + norm * s_vec)

    pltpu.sync_copy(out_v, out.at[pl.ds(base, SAMPLES_PER_W)])


@functools.lru_cache(maxsize=1)
def _built_kernels():
    return _build_segsum_kernel(), _build_combine_kernel()


@jax.jit
def kernel(user_ids, item_ids, hist_items, hist_segments, user_emb,
           item_emb, impl_emb, user_bias, item_bias, global_bias):
    segsum_kernel, combine_kernel = _built_kernels()
    partial, counts = segsum_kernel(hist_items.reshape(T // K, K),
                                    hist_segments.reshape(T // K, K),
                                    impl_emb)
    res = combine_kernel(user_ids, item_ids, user_emb, item_emb,
                         user_bias.reshape(U), item_bias.reshape(I),
                         partial, counts)
    return res + global_bias[0]


# speedup vs baseline: 3.4955x; 3.4955x over previous
import functools

import jax
import jax.numpy as jnp
from jax import lax
from jax.experimental import pallas as pl
from jax.experimental.pallas import tpu as pltpu
from jax.experimental.pallas import tpu_sc as plsc

U = 100000
I = 100000
D = 128
B = 4096
T = 819200

NC = 2
NS = 16
NW = NC * NS
K = 64

ITEMS_PER_W = T // NW
CHUNKS_PER_W = ITEMS_PER_W // K
NBUF = 8
SLACK = 3
NRUN = 8
ROWS_PER_TILE = B // NS
SAMPLES_PER_W = B // NW

def _build_segsum_kernel():
    mesh = plsc.VectorSubcoreMesh(core_axis_name="c", subcore_axis_name="s")
    return functools.partial(
        pl.kernel,
        out_type=[
            jax.ShapeDtypeStruct((NC, B, D), jnp.float32),
            jax.ShapeDtypeStruct((NC, B), jnp.float32),
        ],
        mesh=mesh,
        compiler_params=pltpu.CompilerParams(needs_layout_passes=False),
        scratch_types=[
            pltpu.VMEM((3, NBUF, K), jnp.int32),
            pltpu.VMEM((3, NBUF, K), jnp.int32),
            [pltpu.VMEM((K, D), jnp.float32) for _ in range(NBUF)],
            pltpu.VMEM((2, NRUN, D), jnp.float32),
            pltpu.VMEM((2, NRUN), jnp.int32),
            pltpu.VMEM((ROWS_PER_TILE,), jnp.float32),
            pltpu.VMEM((K,), jnp.float32),
            pltpu.VMEM_SHARED((B, D), jnp.float32),
            pltpu.VMEM_SHARED((B,), jnp.float32),
            [pltpu.SemaphoreType.DMA for _ in range(NBUF)],
            [pltpu.SemaphoreType.DMA for _ in range(NBUF)],
            [pltpu.SemaphoreType.DMA for _ in range(2)],
            pltpu.SemaphoreType.DMA,
            pltpu.SemaphoreType.DMA,
        ],
    )(_segsum_body)


def _segsum_body(hist_items, hist_segments, impl_emb, partial_out,
                 counts_out, items_v, segs_v, rows, runsum_v, runseg_v,
                 zc_v, ones_v, accum_s, counts_s, gsem, csem,
                 rsem, isem, ssem):
    cid = lax.axis_index("c")
    sid = lax.axis_index("s")
    wid = cid * NS + sid
    ngroups = CHUNKS_PER_W // NBUF

    zeros16 = jnp.zeros((16,), jnp.float32)
    izeros16 = jnp.zeros((16,), jnp.int32)
    lanes = lax.iota(jnp.int32, 16)

    def _zero_row(r, _):
        for j in range(D // 16):
            rows[0][r, pl.ds(j * 16, 16)] = zeros16
        return 0

    lax.fori_loop(0, K, _zero_row, 0)
    for j in range(ROWS_PER_TILE // 16):
        zc_v[pl.ds(j * 16, 16)] = zeros16
    for j in range(K // 16):
        ones_v[pl.ds(j * 16, 16)] = jnp.ones((16,), jnp.float32)
    for pp in range(2):
        runseg_v[pp, pl.ds(0, 16)] = izeros16
        for r in range(NRUN):
            for j in range(D // 16):
                runsum_v[pp, r, pl.ds(j * 16, 16)] = zeros16

    row0 = pl.multiple_of(sid * ROWS_PER_TILE, ROWS_PER_TILE)
    for h in range(ROWS_PER_TILE // K):
        pltpu.sync_copy(rows[0], accum_s.at[pl.ds(row0 + h * K, K)])
    pltpu.sync_copy(zc_v, counts_s.at[pl.ds(row0, ROWS_PER_TILE)])
    plsc.subcore_barrier()

    chunk0 = pl.multiple_of(wid * CHUNKS_PER_W, CHUNKS_PER_W)

    def _idx_copies(g, slot):
        src_rows = pl.ds(chunk0 + g * NBUF, NBUF)
        return (pltpu.make_async_copy(hist_items.at[src_rows],
                                      items_v.at[slot], isem),
                pltpu.make_async_copy(hist_segments.at[src_rows],
                                      segs_v.at[slot], ssem))

    def _gather(slot, b, buf):
        return pltpu.make_async_copy(impl_emb.at[items_v.at[slot, b]],
                                     rows[buf], gsem[buf])

    def _count(slot, b):
        return pltpu.make_async_copy(ones_v,
                                     counts_s.at[segs_v.at[slot, b]],
                                     csem[b])

    def _run_scatter(pp):
        return pltpu.make_async_copy(runsum_v.at[pp],
                                     accum_s.at[runseg_v.at[pp]],
                                     rsem[pp])

    def _reduce_and_scatter(g, p, b):
        pp = b & 1
        last_idx = izeros16 + 15
        pp_vec = izeros16 + pp

        vregs = [segs_v[p, b, pl.ds(i * 16, 16)] for i in range(K // 16)]
        seg0 = vregs[0].at[izeros16].get(mode="promise_in_bounds")
        seg1 = vregs[-1].at[last_idx].get(mode="promise_in_bounds")
        ok = jnp.bool_(True)
        for v in vregs:
            ok = jnp.logical_and(
                ok, jnp.all(jnp.logical_or(v == seg0, v == seg1)))

        plsc.store_scatter(runseg_v, [pp_vec, lanes],
                           jnp.where(lanes == 0, seg0, seg1),
                           mask=lanes < 2)

        def _row(r, carry):
            r0 = pl.multiple_of(jnp.bitwise_and(r, -16), 16)
            rvec = segs_v[p, b, pl.ds(r0, 16)]
            sr = rvec.at[izeros16 + jnp.bitwise_and(r, 15)].get(
                mode="promise_in_bounds")
            f0 = jnp.where(sr == seg0, 1.0, 0.0)
            f1 = 1.0 - f0
            out = []
            for j in range(D // 16):
                x = rows[b][r, pl.ds(j * 16, 16)]
                out.append(carry[j] + x * f0)
                out.append(carry[j + D // 16] + x * f1)
            return tuple(out[::2]) + tuple(out[1::2])

        zero_acc = tuple(zeros16 for _ in range(D // 8))
        acc = lax.fori_loop(0, K, _row, zero_acc)

        @pl.when(ok)
        def _():
            for j in range(D // 16):
                runsum_v[pp, 0, pl.ds(j * 16, 16)] = acc[j]
                runsum_v[pp, 1, pl.ds(j * 16, 16)] = acc[j + D // 16]

        @pl.when(jnp.logical_not(ok))
        def _():
            for j in range(D // 16):
                runsum_v[pp, 0, pl.ds(j * 16, 16)] = zeros16
                runsum_v[pp, 1, pl.ds(j * 16, 16)] = zeros16
            pltpu.sync_copy(rows[b], accum_s.at[segs_v.at[p, b]],
                            add=True)

        _run_scatter(pp).start(add=True)

    for c in _idx_copies(0, 0):
        c.start()
    for c in _idx_copies(0, 0):
        c.wait()
    for b in range(NBUF - SLACK):
        _gather(0, b, b).start()
    for c in _idx_copies(1, 1):
        c.start()

    def _group(g, _):
        p = lax.rem(g, 3)
        pn = lax.rem(g + 1, 3)
        pm = lax.rem(g + 2, 3)

        for b in range(NBUF):
            _gather(p, b, b).wait()
            _count(p, b).start(add=True)

            if b >= 2:
                _run_scatter(b & 1).wait()
            else:
                @pl.when(g > 0)
                def _():
                    _run_scatter(b & 1).wait()

            _reduce_and_scatter(g, p, b)

            if b >= SLACK:
                bb = b - SLACK

                @pl.when(g < ngroups - 1)
                def _():
                    _gather(pn, bb, bb).start()
            else:
                bb = b + NBUF - SLACK
                _gather(p, bb, bb).start()

            if b == 1:
                @pl.when(g < ngroups - 1)
                def _():
                    for c in _idx_copies(g + 1, pn):
                        c.wait()

        for b in range(NBUF):
            _count(p, b).wait()

        @pl.when(g < ngroups - 2)
        def _():
            for c in _idx_copies(g + 2, pm):
                c.start()

        return 0

    lax.fori_loop(0, ngroups, _group, 0)
    for pp in range(2):
        _run_scatter(pp).wait()
    plsc.subcore_barrier()

    for h in range(ROWS_PER_TILE // K):
        pltpu.sync_copy(accum_s.at[pl.ds(row0 + h * K, K)],
                        partial_out.at[cid, pl.ds(row0 + h * K, K)])
    pltpu.sync_copy(counts_s.at[pl.ds(row0, ROWS_PER_TILE)],
                    counts_out.at[cid, pl.ds(row0, ROWS_PER_TILE)])


def _build_combine_kernel():
    mesh = plsc.VectorSubcoreMesh(core_axis_name="c", subcore_axis_name="s")
    return functools.partial(
        pl.kernel,
        out_type=jax.ShapeDtypeStruct((B,), jnp.float32),
        mesh=mesh,
        compiler_params=pltpu.CompilerParams(needs_layout_passes=False),
        scratch_types=[
            pltpu.VMEM((SAMPLES_PER_W,), jnp.int32),
            pltpu.VMEM((SAMPLES_PER_W,), jnp.int32),
            pltpu.VMEM((SAMPLES_PER_W, D), jnp.float32),
            pltpu.VMEM((SAMPLES_PER_W, D), jnp.float32),
            pltpu.VMEM((SAMPLES_PER_W, D), jnp.float32),
            pltpu.VMEM((SAMPLES_PER_W, D), jnp.float32),
            pltpu.VMEM((SAMPLES_PER_W,), jnp.float32),
            pltpu.VMEM((SAMPLES_PER_W,), jnp.float32),
            pltpu.VMEM((SAMPLES_PER_W,), jnp.float32),
            pltpu.VMEM((SAMPLES_PER_W,), jnp.float32),
            pltpu.VMEM((SAMPLES_PER_W,), jnp.float32),
            pltpu.SemaphoreType.DMA,
        ],
    )(_combine_body)


def _combine_body(user_ids, item_ids, user_emb, item_emb, user_bias,
                    item_bias, partial, counts, out,
                    uidx_v, iidx_v, ue_v, ie_v, p0_v, p1_v, ub_v, ib_v,
                    c0_v, c1_v, out_v, sem):
    cid = lax.axis_index("c")
    sid = lax.axis_index("s")
    wid = cid * NS + sid
    base = pl.multiple_of(wid * SAMPLES_PER_W, SAMPLES_PER_W)

    pltpu.sync_copy(user_ids.at[pl.ds(base, SAMPLES_PER_W)], uidx_v)
    pltpu.sync_copy(item_ids.at[pl.ds(base, SAMPLES_PER_W)], iidx_v)
    pltpu.async_copy(user_emb.at[uidx_v], ue_v, sem).wait()
    pltpu.async_copy(item_emb.at[iidx_v], ie_v, sem).wait()
    pltpu.async_copy(user_bias.at[uidx_v], ub_v, sem).wait()
    pltpu.async_copy(item_bias.at[iidx_v], ib_v, sem).wait()
    pltpu.sync_copy(partial.at[0, pl.ds(base, SAMPLES_PER_W)], p0_v)
    pltpu.sync_copy(partial.at[1, pl.ds(base, SAMPLES_PER_W)], p1_v)
    pltpu.sync_copy(counts.at[0, pl.ds(base, SAMPLES_PER_W)], c0_v)
    pltpu.sync_copy(counts.at[1, pl.ds(base, SAMPLES_PER_W)], c1_v)

    lanes = lax.iota(jnp.int32, 16)

    for g in range(SAMPLES_PER_W // 16):
        s0 = g * 16
        c = c0_v[pl.ds(s0, 16)] + c1_v[pl.ds(s0, 16)]
        x = jnp.maximum(c, 1.0)
        s = 0.5 * (x + 1.0)
        for _ in range(22):
            s = 0.5 * (s + x / s)
        norm = jnp.where(c > 0.0, 1.0 / s, 0.0)

        def _row(r, carry):
            a_vec, s_vec = carry
            a_acc = jnp.zeros((16,), jnp.float32)
            s_acc = jnp.zeros((16,), jnp.float32)
            row = s0 + r
            for j in range(D // 16):
                sl = pl.ds(j * 16, 16)
                iv = ie_v[row, sl]
                a_acc = a_acc + ue_v[row, sl] * iv
                s_acc = s_acc + (p0_v[row, sl] + p1_v[row, sl]) * iv
            sel = lanes == r
            a_vec = jnp.where(sel, jnp.sum(a_acc), a_vec)
            s_vec = jnp.where(sel, jnp.sum(s_acc), s_vec)
            return a_vec, s_vec

        zero16 = jnp.zeros((16,), jnp.float32)
        a_vec, s_vec = lax.fori_loop(0, 16, _row, (zero16, zero16))
        out_v[pl.ds(s0, 16)] = (ub_v[pl.ds(s0, 16)] + ib_v[pl.ds(s0, 16)]
                                + a_vec + norm * s_vec)

    pltpu.sync_copy(out_v, out.at[pl.ds(base, SAMPLES_PER_W)])


@functools.lru_cache(maxsize=1)
def _built_kernels():
    return _build_segsum_kernel(), _build_combine_kernel()


@jax.jit
def kernel(user_ids, item_ids, hist_items, hist_segments, user_emb,
           item_emb, impl_emb, user_bias, item_bias, global_bias):
    segsum_kernel, combine_kernel = _built_kernels()
    partial, counts = segsum_kernel(hist_items.reshape(T // K, K),
                                    hist_segments.reshape(T // K, K),
                                    impl_emb)
    res = combine_kernel(user_ids, item_ids, user_emb, item_emb,
                         user_bias.reshape(U), item_bias.reshape(I),
                         partial, counts)
    return res + global_bias[0]


# speedup vs baseline: 3.9825x; 1.1393x over previous
import functools

import jax
import jax.numpy as jnp
from jax import lax
from jax.experimental import pallas as pl
from jax.experimental.pallas import tpu as pltpu
from jax.experimental.pallas import tpu_sc as plsc

U = 100000
I = 100000
D = 128
B = 4096
T = 819200

NC = 2
NS = 16
NW = NC * NS
K = 64

ITEMS_PER_W = T // NW
CHUNKS_PER_W = ITEMS_PER_W // K
NBUF = 8
SLACK = 3
NRUN = 8
ROWS_PER_TILE = B // NS
SAMPLES_PER_W = B // NW

def _build_segsum_kernel():
    mesh = plsc.VectorSubcoreMesh(core_axis_name="c", subcore_axis_name="s")
    return functools.partial(
        pl.kernel,
        out_type=[
            jax.ShapeDtypeStruct((NC, B, D), jnp.float32),
            jax.ShapeDtypeStruct((NC, B), jnp.float32),
        ],
        mesh=mesh,
        compiler_params=pltpu.CompilerParams(needs_layout_passes=False),
        scratch_types=[
            pltpu.VMEM((3, NBUF, K), jnp.int32),
            pltpu.VMEM((3, NBUF, K), jnp.int32),
            [pltpu.VMEM((K, D), jnp.float32) for _ in range(NBUF)],
            pltpu.VMEM((2, NRUN, D), jnp.float32),
            pltpu.VMEM((2, NRUN), jnp.int32),
            pltpu.VMEM((ROWS_PER_TILE,), jnp.float32),
            pltpu.VMEM((K,), jnp.float32),
            pltpu.VMEM_SHARED((B, D), jnp.float32),
            pltpu.VMEM_SHARED((B,), jnp.float32),
            [pltpu.SemaphoreType.DMA for _ in range(NBUF)],
            [pltpu.SemaphoreType.DMA for _ in range(NBUF)],
            [pltpu.SemaphoreType.DMA for _ in range(2)],
            pltpu.SemaphoreType.DMA,
            pltpu.SemaphoreType.DMA,
        ],
    )(_segsum_body)


def _segsum_body(hist_items, hist_segments, impl_emb, partial_out,
                 counts_out, items_v, segs_v, rows, runsum_v, runseg_v,
                 zc_v, ones_v, accum_s, counts_s, gsem, csem,
                 rsem, isem, ssem):
    cid = lax.axis_index("c")
    sid = lax.axis_index("s")
    wid = cid * NS + sid
    ngroups = CHUNKS_PER_W // NBUF

    zeros16 = jnp.zeros((16,), jnp.float32)
    izeros16 = jnp.zeros((16,), jnp.int32)
    lanes = lax.iota(jnp.int32, 16)

    def _zero_row(r, _):
        for j in range(D // 16):
            rows[0][r, pl.ds(j * 16, 16)] = zeros16
        return 0

    lax.fori_loop(0, K, _zero_row, 0)
    for j in range(ROWS_PER_TILE // 16):
        zc_v[pl.ds(j * 16, 16)] = zeros16
    for j in range(K // 16):
        ones_v[pl.ds(j * 16, 16)] = jnp.ones((16,), jnp.float32)
    for pp in range(2):
        runseg_v[pp, pl.ds(0, 16)] = izeros16
        for r in range(NRUN):
            for j in range(D // 16):
                runsum_v[pp, r, pl.ds(j * 16, 16)] = zeros16

    row0 = pl.multiple_of(sid * ROWS_PER_TILE, ROWS_PER_TILE)
    for h in range(ROWS_PER_TILE // K):
        pltpu.sync_copy(rows[0], accum_s.at[pl.ds(row0 + h * K, K)])
    pltpu.sync_copy(zc_v, counts_s.at[pl.ds(row0, ROWS_PER_TILE)])
    plsc.subcore_barrier()

    chunk0 = pl.multiple_of(wid * CHUNKS_PER_W, CHUNKS_PER_W)

    def _idx_copies(g, slot):
        src_rows = pl.ds(chunk0 + g * NBUF, NBUF)
        return (pltpu.make_async_copy(hist_items.at[src_rows],
                                      items_v.at[slot], isem),
                pltpu.make_async_copy(hist_segments.at[src_rows],
                                      segs_v.at[slot], ssem))

    def _gather(slot, b, buf):
        return pltpu.make_async_copy(impl_emb.at[items_v.at[slot, b]],
                                     rows[buf], gsem[buf])

    def _count(slot, b):
        return pltpu.make_async_copy(ones_v,
                                     counts_s.at[segs_v.at[slot, b]],
                                     csem[b])

    def _run_scatter(pp):
        return pltpu.make_async_copy(runsum_v.at[pp],
                                     accum_s.at[runseg_v.at[pp]],
                                     rsem[pp])

    def _reduce_and_scatter(g, p, b):
        pp = b & 1
        last_idx = izeros16 + 15
        pp_vec = izeros16 + pp

        vregs = [segs_v[p, b, pl.ds(i * 16, 16)] for i in range(K // 16)]
        seg0 = vregs[0].at[izeros16].get(mode="promise_in_bounds")
        seg1 = vregs[-1].at[last_idx].get(mode="promise_in_bounds")
        one_seg = jnp.bool_(True)
        two_seg = jnp.bool_(True)
        for v in vregs:
            one_seg = jnp.logical_and(one_seg, jnp.all(v == seg0))
            two_seg = jnp.logical_and(
                two_seg, jnp.all(jnp.logical_or(v == seg0, v == seg1)))

        plsc.store_scatter(runseg_v, [pp_vec, lanes],
                           jnp.where(lanes == 0, seg0, seg1),
                           mask=lanes < 2)

        @pl.when(one_seg)
        def _():
            def _row1(r, carry):
                out = []
                for j in range(D // 16):
                    out.append(carry[j] + rows[b][r, pl.ds(j * 16, 16)])
                return tuple(out)

            acc = lax.fori_loop(0, K, _row1,
                                tuple(zeros16 for _ in range(D // 16)))
            for j in range(D // 16):
                runsum_v[pp, 0, pl.ds(j * 16, 16)] = acc[j]
                runsum_v[pp, 1, pl.ds(j * 16, 16)] = zeros16

        @pl.when(jnp.logical_and(two_seg, jnp.logical_not(one_seg)))
        def _():
            def _row(r, carry):
                r0 = pl.multiple_of(jnp.bitwise_and(r, -16), 16)
                rvec = segs_v[p, b, pl.ds(r0, 16)]
                sr = rvec.at[izeros16 + jnp.bitwise_and(r, 15)].get(
                    mode="promise_in_bounds")
                f0 = jnp.where(sr == seg0, 1.0, 0.0)
                f1 = 1.0 - f0
                out = []
                for j in range(D // 16):
                    x = rows[b][r, pl.ds(j * 16, 16)]
                    out.append(carry[j] + x * f0)
                    out.append(carry[j + D // 16] + x * f1)
                return tuple(out[::2]) + tuple(out[1::2])

            zero_acc = tuple(zeros16 for _ in range(D // 8))
            acc = lax.fori_loop(0, K, _row, zero_acc)
            for j in range(D // 16):
                runsum_v[pp, 0, pl.ds(j * 16, 16)] = acc[j]
                runsum_v[pp, 1, pl.ds(j * 16, 16)] = acc[j + D // 16]

        @pl.when(jnp.logical_not(two_seg))
        def _():
            for j in range(D // 16):
                runsum_v[pp, 0, pl.ds(j * 16, 16)] = zeros16
                runsum_v[pp, 1, pl.ds(j * 16, 16)] = zeros16
            pltpu.sync_copy(rows[b], accum_s.at[segs_v.at[p, b]],
                            add=True)

        _run_scatter(pp).start(add=True)

    for c in _idx_copies(0, 0):
        c.start()
    for c in _idx_copies(0, 0):
        c.wait()
    for b in range(NBUF - SLACK):
        _gather(0, b, b).start()
    for c in _idx_copies(1, 1):
        c.start()

    def _group(g, _):
        p = lax.rem(g, 3)
        pn = lax.rem(g + 1, 3)
        pm = lax.rem(g + 2, 3)

        for b in range(NBUF):
            _gather(p, b, b).wait()
            _count(p, b).start(add=True)

            if b >= 2:
                _run_scatter(b & 1).wait()
            else:
                @pl.when(g > 0)
                def _():
                    _run_scatter(b & 1).wait()

            _reduce_and_scatter(g, p, b)

            if b >= SLACK:
                bb = b - SLACK

                @pl.when(g < ngroups - 1)
                def _():
                    _gather(pn, bb, bb).start()
            else:
                bb = b + NBUF - SLACK
                _gather(p, bb, bb).start()

            if b == 1:
                @pl.when(g < ngroups - 1)
                def _():
                    for c in _idx_copies(g + 1, pn):
                        c.wait()

        for b in range(NBUF):
            _count(p, b).wait()

        @pl.when(g < ngroups - 2)
        def _():
            for c in _idx_copies(g + 2, pm):
                c.start()

        return 0

    lax.fori_loop(0, ngroups, _group, 0)
    for pp in range(2):
        _run_scatter(pp).wait()
    plsc.subcore_barrier()

    for h in range(ROWS_PER_TILE // K):
        pltpu.sync_copy(accum_s.at[pl.ds(row0 + h * K, K)],
                        partial_out.at[cid, pl.ds(row0 + h * K, K)])
    pltpu.sync_copy(counts_s.at[pl.ds(row0, ROWS_PER_TILE)],
                    counts_out.at[cid, pl.ds(row0, ROWS_PER_TILE)])


def _build_combine_kernel():
    mesh = plsc.VectorSubcoreMesh(core_axis_name="c", subcore_axis_name="s")
    return functools.partial(
        pl.kernel,
        out_type=jax.ShapeDtypeStruct((B,), jnp.float32),
        mesh=mesh,
        compiler_params=pltpu.CompilerParams(needs_layout_passes=False),
        scratch_types=[
            pltpu.VMEM((SAMPLES_PER_W,), jnp.int32),
            pltpu.VMEM((SAMPLES_PER_W,), jnp.int32),
            pltpu.VMEM((SAMPLES_PER_W, D), jnp.float32),
            pltpu.VMEM((SAMPLES_PER_W, D), jnp.float32),
            pltpu.VMEM((SAMPLES_PER_W, D), jnp.float32),
            pltpu.VMEM((SAMPLES_PER_W, D), jnp.float32),
            pltpu.VMEM((SAMPLES_PER_W,), jnp.float32),
            pltpu.VMEM((SAMPLES_PER_W,), jnp.float32),
            pltpu.VMEM((SAMPLES_PER_W,), jnp.float32),
            pltpu.VMEM((SAMPLES_PER_W,), jnp.float32),
            pltpu.VMEM((SAMPLES_PER_W,), jnp.float32),
            pltpu.SemaphoreType.DMA,
        ],
    )(_combine_body)


def _combine_body(user_ids, item_ids, user_emb, item_emb, user_bias,
                    item_bias, partial, counts, out,
                    uidx_v, iidx_v, ue_v, ie_v, p0_v, p1_v, ub_v, ib_v,
                    c0_v, c1_v, out_v, sem):
    cid = lax.axis_index("c")
    sid = lax.axis_index("s")
    wid = cid * NS + sid
    base = pl.multiple_of(wid * SAMPLES_PER_W, SAMPLES_PER_W)

    pltpu.sync_copy(user_ids.at[pl.ds(base, SAMPLES_PER_W)], uidx_v)
    pltpu.sync_copy(item_ids.at[pl.ds(base, SAMPLES_PER_W)], iidx_v)
    pltpu.async_copy(user_emb.at[uidx_v], ue_v, sem).wait()
    pltpu.async_copy(item_emb.at[iidx_v], ie_v, sem).wait()
    pltpu.async_copy(user_bias.at[uidx_v], ub_v, sem).wait()
    pltpu.async_copy(item_bias.at[iidx_v], ib_v, sem).wait()
    pltpu.sync_copy(partial.at[0, pl.ds(base, SAMPLES_PER_W)], p0_v)
    pltpu.sync_copy(partial.at[1, pl.ds(base, SAMPLES_PER_W)], p1_v)
    pltpu.sync_copy(counts.at[0, pl.ds(base, SAMPLES_PER_W)], c0_v)
    pltpu.sync_copy(counts.at[1, pl.ds(base, SAMPLES_PER_W)], c1_v)

    lanes = lax.iota(jnp.int32, 16)

    for g in range(SAMPLES_PER_W // 16):
        s0 = g * 16
        c = c0_v[pl.ds(s0, 16)] + c1_v[pl.ds(s0, 16)]
        x = jnp.maximum(c, 1.0)
        s = 0.5 * (x + 1.0)
        for _ in range(22):
            s = 0.5 * (s + x / s)
        norm = jnp.where(c > 0.0, 1.0 / s, 0.0)

        def _row(r, carry):
            a_vec, s_vec = carry
            a_acc = jnp.zeros((16,), jnp.float32)
            s_acc = jnp.zeros((16,), jnp.float32)
            row = s0 + r
            for j in range(D // 16):
                sl = pl.ds(j * 16, 16)
                iv = ie_v[row, sl]
                a_acc = a_acc + ue_v[row, sl] * iv
                s_acc = s_acc + (p0_v[row, sl] + p1_v[row, sl]) * iv
            sel = lanes == r
            a_vec = jnp.where(sel, jnp.sum(a_acc), a_vec)
            s_vec = jnp.where(sel, jnp.sum(s_acc), s_vec)
            return a_vec, s_vec

        zero16 = jnp.zeros((16,), jnp.float32)
        a_vec, s_vec = lax.fori_loop(0, 16, _row, (zero16, zero16))
        out_v[pl.ds(s0, 16)] = (ub_v[pl.ds(s0, 16)] + ib_v[pl.ds(s0, 16)]
                                + a_vec + norm * s_vec)

    pltpu.sync_copy(out_v, out.at[pl.ds(base, SAMPLES_PER_W)])


@functools.lru_cache(maxsize=1)
def _built_kernels():
    return _build_segsum_kernel(), _build_combine_kernel()


@jax.jit
def kernel(user_ids, item_ids, hist_items, hist_segments, user_emb,
           item_emb, impl_emb, user_bias, item_bias, global_bias):
    segsum_kernel, combine_kernel = _built_kernels()
    partial, counts = segsum_kernel(hist_items.reshape(T // K, K),
                                    hist_segments.reshape(T // K, K),
                                    impl_emb)
    res = combine_kernel(user_ids, item_ids, user_emb, item_emb,
                         user_bias.reshape(U), item_bias.reshape(I),
                         partial, counts)
    return res + global_bias[0]
